# baseline (speedup 1.0000x reference)
import functools

import jax
import jax.numpy as jnp
from jax import lax
from jax.experimental import pallas as pl
from jax.experimental.pallas import tpu as pltpu
from jax.experimental.pallas import tpu_sc as plsc

VOCAB = 100000
EMBED = 64
B = 1024
L = 50

NW = 32
EX_PER_W = B // NW
LANES = 16

BBLK = 32
VS = 2048
NV = pl.cdiv(VOCAB, VS)
TAIL = VOCAB - (NV - 1) * VS
NB = B // BBLK


def _sc_pool_body(ctx_hbm, emb_hbm, out_hbm, idx_v, rows_v, pooled_v, sem):
    wid = lax.axis_index("s") * 2 + lax.axis_index("c")
    pltpu.sync_copy(ctx_hbm.at[pl.ds(wid * EX_PER_W, EX_PER_W), :], idx_v)
    copies = [
        pltpu.async_copy(
            emb_hbm.at[idx_v.at[e]],
            rows_v.at[pl.ds(e * L, L), :],
            sem,
        )
        for e in range(EX_PER_W)
    ]
    for cp in copies:
        cp.wait()

    def body(e, carry):
        base = e * L
        for c in range(EMBED // LANES):
            acc = rows_v[base, pl.ds(c * LANES, LANES)]
            for l in range(1, L):
                acc = acc + rows_v[base + l, pl.ds(c * LANES, LANES)]
            pooled_v[e, pl.ds(c * LANES, LANES)] = acc
        return carry

    lax.fori_loop(0, EX_PER_W, body, 0)
    pltpu.sync_copy(pooled_v, out_hbm.at[pl.ds(wid * EX_PER_W, EX_PER_W), :])


_sc_pool = functools.partial(
    pl.kernel,
    out_type=jax.ShapeDtypeStruct((B, EMBED), jnp.float32),
    mesh=plsc.VectorSubcoreMesh(core_axis_name="c", subcore_axis_name="s"),
    scratch_types=[
        pltpu.VMEM((EX_PER_W, L), jnp.int32),
        pltpu.VMEM((EX_PER_W * L, EMBED), jnp.float32),
        pltpu.VMEM((EX_PER_W, EMBED), jnp.float32),
        pltpu.SemaphoreType.DMA,
    ],
    compiler_params=pltpu.CompilerParams(use_tc_tiling_on_sc=False),
)(_sc_pool_body)


def _mm_body(c_ref, p_ref, w_ref, b_ref, o_ref, wt_vmem):
    bi = pl.program_id(0)
    vi = pl.program_id(1)

    @pl.when(bi == 0)
    def _():
        wt_vmem[:, pl.ds(vi * VS, VS)] = w_ref[:].T

    x = p_ref[:] / c_ref[:]
    res = lax.dot_general(
        x, wt_vmem[:, pl.ds(vi * VS, VS)], (((1,), (0,)), ((), ())),
        preferred_element_type=jnp.float32,
    )

    @pl.when(vi < NV - 1)
    def _():
        o_ref[:, pl.ds(vi * VS, VS)] = res + b_ref[:, pl.ds(vi * VS, VS)]

    @pl.when(vi == NV - 1)
    def _():
        o_ref[:, pl.ds((NV - 1) * VS, TAIL)] = (
            res[:, :TAIL] + b_ref[:, pl.ds((NV - 1) * VS, TAIL)]
        )


def kernel(context_indices, actual_C, embedding, W, b):
    pooled = _sc_pool(context_indices, embedding)

    c2d = actual_C.astype(jnp.float32).reshape(B, 1)
    b2d = b.reshape(1, VOCAB)
    scores = pl.pallas_call(
        _mm_body,
        grid=(NB, NV),
        in_specs=[
            pl.BlockSpec((BBLK, 1), lambda bi, vi: (bi, 0)),
            pl.BlockSpec((BBLK, EMBED), lambda bi, vi: (bi, 0)),
            pl.BlockSpec(
                (VS, EMBED),
                lambda bi, vi: (jnp.where(bi == 0, vi, NV - 1), 0),
            ),
            pl.BlockSpec((1, VOCAB), lambda bi, vi: (0, 0)),
        ],
        out_specs=pl.BlockSpec((BBLK, VOCAB), lambda bi, vi: (bi, 0)),
        out_shape=jax.ShapeDtypeStruct((B, VOCAB), jnp.float32),
        scratch_shapes=[
            pltpu.VMEM((EMBED, NV * VS), jnp.float32),
        ],
        compiler_params=pltpu.CompilerParams(
            vmem_limit_bytes=62 * 1024 * 1024,
        ),
    )(c2d, pooled, W, b2d)
    return scores


# speedup vs baseline: 4.1269x; 4.1269x over previous
import functools

import jax
import jax.numpy as jnp
from jax import lax
from jax.experimental import pallas as pl
from jax.experimental.pallas import tpu as pltpu
from jax.experimental.pallas import tpu_sc as plsc

VOCAB = 100000
EMBED = 64
B = 1024
L = 50

NW = 32
EX_PER_W = B // NW
LANES = 16

RB = 32
NB = B // RB
VS = 4096
NV = pl.cdiv(VOCAB, VS)
VPAD = NV * VS
TAIL = VOCAB - (NV - 1) * VS


def _sc_pool_body(ctx_hbm, emb_hbm, out_hbm, idx_v, rows_v, pooled_v, sem):
    wid = lax.axis_index("s") * 2 + lax.axis_index("c")
    pltpu.sync_copy(ctx_hbm.at[pl.ds(wid * EX_PER_W, EX_PER_W), :], idx_v)
    copies = [
        pltpu.async_copy(
            emb_hbm.at[idx_v.at[e]],
            rows_v.at[pl.ds(e * L, L), :],
            sem,
        )
        for e in range(EX_PER_W)
    ]
    for cp in copies:
        cp.wait()

    def body(e, carry):
        base = e * L
        for c in range(EMBED // LANES):
            acc = rows_v[base, pl.ds(c * LANES, LANES)]
            for l in range(1, L):
                acc = acc + rows_v[base + l, pl.ds(c * LANES, LANES)]
            pooled_v[e, pl.ds(c * LANES, LANES)] = acc
        return carry

    lax.fori_loop(0, EX_PER_W, body, 0)
    pltpu.sync_copy(pooled_v, out_hbm.at[pl.ds(wid * EX_PER_W, EX_PER_W), :])


_sc_pool = functools.partial(
    pl.kernel,
    out_type=jax.ShapeDtypeStruct((B, EMBED), jnp.float32),
    mesh=plsc.VectorSubcoreMesh(core_axis_name="c", subcore_axis_name="s"),
    scratch_types=[
        pltpu.VMEM((EX_PER_W, L), jnp.int32),
        pltpu.VMEM((EX_PER_W * L, EMBED), jnp.float32),
        pltpu.VMEM((EX_PER_W, EMBED), jnp.float32),
        pltpu.SemaphoreType.DMA,
    ],
    compiler_params=pltpu.CompilerParams(use_tc_tiling_on_sc=False),
)(_sc_pool_body)


def _mm_body(c_ref, p_ref, wt_ref, bt_ref, o_ref):
    x = p_ref[:] / c_ref[:]
    o_ref[:] = (
        lax.dot_general(
            wt_ref[:], x, (((0,), (1,)), ((), ())),
            preferred_element_type=jnp.float32,
        )
        + bt_ref[:]
    )


def kernel(context_indices, actual_C, embedding, W, b):
    pooled = _sc_pool(context_indices, embedding)

    c2d = actual_C.astype(jnp.float32).reshape(B, 1)
    bt2d = b.reshape(VOCAB, 1)
    wt = W.T
    scores_t = pl.pallas_call(
        _mm_body,
        grid=(NV,),
        in_specs=[
            pl.BlockSpec((B, 1), lambda vi: (0, 0)),
            pl.BlockSpec((B, EMBED), lambda vi: (0, 0)),
            pl.BlockSpec((EMBED, VS), lambda vi: (0, vi)),
            pl.BlockSpec((VS, 1), lambda vi: (vi, 0)),
        ],
        out_specs=pl.BlockSpec((VS, B), lambda vi: (vi, 0)),
        out_shape=jax.ShapeDtypeStruct((VOCAB, B), jnp.float32),
        compiler_params=pltpu.CompilerParams(
            vmem_limit_bytes=63 * 1024 * 1024,
        ),
    )(c2d, pooled, wt, bt2d)
    return scores_t.T
